# baseline (speedup 1.0000x reference)
import functools

import jax
import jax.numpy as jnp
from jax import lax
from jax.experimental import pallas as pl
from jax.experimental.pallas import tpu as pltpu
from jax.experimental.pallas import tpu_sc as plsc

B = 16384
D = 32
NC = 2
NS = 16
NW = NC * NS
BPW = B // NW
CHUNK = 128
NCHUNK = BPW // CHUNK

MIN_R = 0.5
MAX_R = 5.0

_mesh = plsc.VectorSubcoreMesh(core_axis_name="c", subcore_axis_name="s")


def _sc_gather_body(uid_hbm, mid_hbm, uemb_hbm, memb_hbm, ub_hbm, mb_hbm,
                    uo_hbm, mo_hbm, ubo_hbm, mbo_hbm,
                    uidx_v, midx_v, urows_v, mrows_v, ubias_v, mbias_v, sem):
    wid = lax.axis_index("s") * NC + lax.axis_index("c")
    base = wid * BPW
    pltpu.sync_copy(uid_hbm.at[pl.ds(base, BPW)], uidx_v)
    pltpu.sync_copy(mid_hbm.at[pl.ds(base, BPW)], midx_v)
    for j in range(NCHUNK):
        isl = pl.ds(j * CHUNK, CHUNK)
        osl = pl.ds(base + j * CHUNK, CHUNK)
        pltpu.async_copy(uemb_hbm.at[uidx_v.at[isl]], urows_v, sem).wait()
        pltpu.sync_copy(urows_v, uo_hbm.at[osl])
        pltpu.async_copy(memb_hbm.at[midx_v.at[isl]], mrows_v, sem).wait()
        pltpu.sync_copy(mrows_v, mo_hbm.at[osl])
        pltpu.async_copy(ub_hbm.at[uidx_v.at[isl]], ubias_v, sem).wait()
        pltpu.sync_copy(ubias_v, ubo_hbm.at[osl])
        pltpu.async_copy(mb_hbm.at[midx_v.at[isl]], mbias_v, sem).wait()
        pltpu.sync_copy(mbias_v, mbo_hbm.at[osl])


@jax.jit
def _sc_gather(user_ids, movie_ids, user_embedding, movie_embedding,
               user_biases, movie_biases):
    f32 = jnp.float32
    kern = pl.kernel(
        _sc_gather_body,
        out_type=(
            jax.ShapeDtypeStruct((B, D), f32),
            jax.ShapeDtypeStruct((B, D), f32),
            jax.ShapeDtypeStruct((B,), f32),
            jax.ShapeDtypeStruct((B,), f32),
        ),
        mesh=_mesh,
        compiler_params=pltpu.CompilerParams(use_tc_tiling_on_sc=False),
        scratch_types=[
            pltpu.VMEM((BPW,), jnp.int32),
            pltpu.VMEM((BPW,), jnp.int32),
            pltpu.VMEM((CHUNK, D), f32),
            pltpu.VMEM((CHUNK, D), f32),
            pltpu.VMEM((CHUNK,), f32),
            pltpu.VMEM((CHUNK,), f32),
            pltpu.SemaphoreType.DMA,
        ],
    )
    return kern(user_ids, movie_ids, user_embedding, movie_embedding,
                user_biases.reshape(-1), movie_biases.reshape(-1))


TC_BLK = 2048


def _tc_compute_body(u_ref, m_ref, ub_ref, mb_ref, o_ref):
    dot = jnp.sum(u_ref[...] * m_ref[...], axis=1)
    pred = dot + ub_ref[...] + mb_ref[...]
    o_ref[...] = jnp.clip(pred, MIN_R, MAX_R)


@jax.jit
def _tc_compute(u_rows, m_rows, u_bias, m_bias):
    return pl.pallas_call(
        _tc_compute_body,
        grid=(B // TC_BLK,),
        in_specs=[
            pl.BlockSpec((TC_BLK, D), lambda i: (i, 0)),
            pl.BlockSpec((TC_BLK, D), lambda i: (i, 0)),
            pl.BlockSpec((TC_BLK,), lambda i: (i,)),
            pl.BlockSpec((TC_BLK,), lambda i: (i,)),
        ],
        out_specs=pl.BlockSpec((TC_BLK,), lambda i: (i,)),
        out_shape=jax.ShapeDtypeStruct((B,), jnp.float32),
    )(u_rows, m_rows, u_bias, m_bias)


def kernel(user_ids, movie_ids, user_embedding, movie_embedding,
           user_biases, movie_biases):
    uid = user_ids.astype(jnp.int32)
    mid = movie_ids.astype(jnp.int32)
    u_rows, m_rows, ub, mb = _sc_gather(
        uid, mid, user_embedding, movie_embedding, user_biases, movie_biases)
    return _tc_compute(u_rows, m_rows, ub, mb)


# speedup vs baseline: 1.0513x; 1.0513x over previous
import functools

import jax
import jax.numpy as jnp
from jax import lax
from jax.experimental import pallas as pl
from jax.experimental.pallas import tpu as pltpu
from jax.experimental.pallas import tpu_sc as plsc

B = 16384
D = 32
PK = 128
NC = 2
NS = 16
NW = NC * NS
BPW = B // NW
CHUNK = 128
NCHUNK = BPW // CHUNK
L = 16

MIN_R = 0.5
MAX_R = 5.0

_mesh = plsc.VectorSubcoreMesh(core_axis_name="c", subcore_axis_name="s")


def _worker_id():
    return lax.axis_index("s") * NC + lax.axis_index("c")


def _sc_bias_body(uid_hbm, mid_hbm, ub_hbm, mb_hbm, bs_hbm,
                  uidx_v, midx_v, ubias_v, mbias_v, bsum_v, sem):
    wid = _worker_id()
    base = wid * BPW
    pltpu.sync_copy(uid_hbm.at[pl.ds(base, BPW)], uidx_v)
    pltpu.sync_copy(mid_hbm.at[pl.ds(base, BPW)], midx_v)
    for j in range(NCHUNK):
        isl = pl.ds(j * CHUNK, CHUNK)
        cp_u = pltpu.async_copy(ub_hbm.at[uidx_v.at[isl]], ubias_v, sem)
        cp_m = pltpu.async_copy(mb_hbm.at[midx_v.at[isl]], mbias_v, sem)
        cp_u.wait()
        cp_m.wait()

        @pl.loop(0, CHUNK, step=L)
        def _(k):
            sl = pl.ds(k, L)
            bsum_v[sl] = ubias_v[sl] + mbias_v[sl]

        pltpu.sync_copy(bsum_v, bs_hbm.at[pl.ds(base + j * CHUNK, CHUNK)])


@jax.jit
def _sc_bias(user_ids, movie_ids, user_biases, movie_biases):
    f32 = jnp.float32
    kern = pl.kernel(
        _sc_bias_body,
        out_type=jax.ShapeDtypeStruct((B,), f32),
        mesh=_mesh,
        compiler_params=pltpu.CompilerParams(use_tc_tiling_on_sc=False),
        scratch_types=[
            pltpu.VMEM((BPW,), jnp.int32),
            pltpu.VMEM((BPW,), jnp.int32),
            pltpu.VMEM((CHUNK,), f32),
            pltpu.VMEM((CHUNK,), f32),
            pltpu.VMEM((CHUNK,), f32),
            pltpu.SemaphoreType.DMA,
        ],
    )
    return kern(user_ids, movie_ids,
                user_biases.reshape(-1), movie_biases.reshape(-1))


def _sc_emb_body(uid_hbm, mid_hbm, upk_hbm, mpk_hbm, bs_hbm, out_hbm,
                 uidx_v, midx_v, gu_v, gm_v, cbu_v, cbm_v,
                 upkd_v, mpkd_v, bsum_v, out_v, st_v, sem):
    wid = _worker_id()
    base = wid * BPW
    pltpu.sync_copy(uid_hbm.at[pl.ds(base, BPW)], uidx_v)
    pltpu.sync_copy(mid_hbm.at[pl.ds(base, BPW)], midx_v)

    @pl.loop(0, BPW, step=L)
    def _(k):
        sl = pl.ds(k, L)
        u = uidx_v[sl]
        m = midx_v[sl]
        gu_v[sl] = lax.shift_right_logical(u, 2)
        gm_v[sl] = lax.shift_right_logical(m, 2)
        cbu_v[sl] = lax.shift_left(jnp.bitwise_and(u, 3), 5)
        cbm_v[sl] = lax.shift_left(jnp.bitwise_and(m, 3), 5)

    lane = lax.iota(jnp.int32, L)
    off17 = lane * 17

    for j in range(NCHUNK):
        isl = pl.ds(j * CHUNK, CHUNK)
        cp_b = pltpu.async_copy(bs_hbm.at[pl.ds(base + j * CHUNK, CHUNK)],
                                bsum_v, sem)
        cp_u = pltpu.async_copy(upk_hbm.at[gu_v.at[isl]], upkd_v, sem)
        cp_m = pltpu.async_copy(mpk_hbm.at[gm_v.at[isl]], mpkd_v, sem)
        cp_b.wait()
        cp_u.wait()
        cp_m.wait()

        @pl.loop(0, CHUNK, step=L)
        def _(k):
            cbu16 = cbu_v[pl.ds(j * CHUNK + k, L)]
            cbm16 = cbm_v[pl.ds(j * CHUNK + k, L)]
            for e in range(L):
                cbu = cbu16[e]
                cbm = cbm16[e]
                r = k + e
                s = (upkd_v[r, pl.ds(cbu, L)] * mpkd_v[r, pl.ds(cbm, L)]
                     + upkd_v[r, pl.ds(cbu + L, L)]
                     * mpkd_v[r, pl.ds(cbm + L, L)])
                st_v[pl.ds(e * 17, L)] = s

            acc = bsum_v[pl.ds(k, L)]
            for d in range(L):
                acc = acc + plsc.load_gather(st_v, [off17 + d])
            acc = jnp.minimum(jnp.maximum(acc, MIN_R), MAX_R)
            out_v[pl.ds(k, L)] = acc

        pltpu.sync_copy(out_v, out_hbm.at[pl.ds(base + j * CHUNK, CHUNK)])


@jax.jit
def _sc_emb(user_ids, movie_ids, user_embedding, movie_embedding, bias_sum):
    f32 = jnp.float32
    i32 = jnp.int32
    kern = pl.kernel(
        _sc_emb_body,
        out_type=jax.ShapeDtypeStruct((B,), f32),
        mesh=_mesh,
        compiler_params=pltpu.CompilerParams(needs_layout_passes=False),
        scratch_types=[
            pltpu.VMEM((BPW,), i32),
            pltpu.VMEM((BPW,), i32),
            pltpu.VMEM((BPW,), i32),
            pltpu.VMEM((BPW,), i32),
            pltpu.VMEM((BPW,), i32),
            pltpu.VMEM((BPW,), i32),
            pltpu.VMEM((CHUNK, PK), f32),
            pltpu.VMEM((CHUNK, PK), f32),
            pltpu.VMEM((CHUNK,), f32),
            pltpu.VMEM((CHUNK,), f32),
            pltpu.VMEM((L * 17,), f32),
            pltpu.SemaphoreType.DMA,
        ],
    )
    upk = user_embedding.reshape(-1, PK)
    mpk = movie_embedding.reshape(-1, PK)
    return kern(user_ids, movie_ids, upk, mpk, bias_sum)


def kernel(user_ids, movie_ids, user_embedding, movie_embedding,
           user_biases, movie_biases):
    uid = user_ids.astype(jnp.int32)
    mid = movie_ids.astype(jnp.int32)
    bias_sum = _sc_bias(uid, mid, user_biases, movie_biases)
    return _sc_emb(uid, mid, user_embedding, movie_embedding, bias_sum)
